# baseline (speedup 1.0000x reference)
import functools

import jax
import jax.numpy as jnp
from jax import lax
from jax.experimental import pallas as pl
from jax.experimental.pallas import tpu as pltpu
from jax.experimental.pallas import tpu_sc as plsc

NUM_EMBEDDINGS = 1000000
EMBED_DIM = 64
DENSE_OUT = 64
BATCH = 16384
HIST = 50

NC = 2
NS = 16
NW = NC * NS

BAGS_PER_TILE = BATCH // NW
CHUNK_BAGS = 8
CHUNKS_PER_TILE = BAGS_PER_TILE // CHUNK_BAGS
GHIST = 56
IDX_PER_CHUNK = CHUNK_BAGS * GHIST
XPAD = 128
SUPER_BAGS = 64
SUPER_PER_TILE = BAGS_PER_TILE // SUPER_BAGS
CHUNKS_PER_SUPER = SUPER_BAGS // CHUNK_BAGS


def _sc_pool(xp, table):
    mesh = plsc.VectorSubcoreMesh(core_axis_name="c", subcore_axis_name="s")

    @functools.partial(
        pl.kernel,
        mesh=mesh,
        compiler_params=pltpu.CompilerParams(use_tc_tiling_on_sc=False),
        out_type=jax.ShapeDtypeStruct((BATCH, EMBED_DIM), jnp.float32),
        scratch_types=[
            pltpu.VMEM((2, SUPER_BAGS, XPAD), jnp.int32),
            pltpu.VMEM((2, IDX_PER_CHUNK, EMBED_DIM), jnp.float32),
            pltpu.VMEM((BAGS_PER_TILE, EMBED_DIM), jnp.float32),
            pltpu.SemaphoreType.DMA,
            pltpu.SemaphoreType.DMA,
        ],
    )
    def sc_pool(x_hbm, table_hbm, out_hbm, idx_v, rows_v, acc_v, sem0, sem1):
        wid = lax.axis_index("s") * NC + lax.axis_index("c")
        bag0 = wid * BAGS_PER_TILE
        sems = (sem0, sem1)
        rows_b = (rows_v.at[0], rows_v.at[1])

        def _bag_idx(chunk, j):
            s = chunk // CHUNKS_PER_SUPER
            r = (chunk % CHUNKS_PER_SUPER) * CHUNK_BAGS + j
            return idx_v.at[s % 2, r, pl.ds(0, GHIST)]

        def fire(chunk, b):
            @pl.when(chunk % CHUNKS_PER_SUPER == 0)
            def _():
                s = chunk // CHUNKS_PER_SUPER
                pltpu.sync_copy(
                    x_hbm.at[pl.ds(bag0 + s * SUPER_BAGS, SUPER_BAGS)],
                    idx_v.at[s % 2])

            for j in range(CHUNK_BAGS):
                pltpu.async_copy(
                    table_hbm.at[_bag_idx(chunk, j)],
                    rows_b[b].at[pl.ds(j * GHIST, GHIST)],
                    sems[b])

        def drain(chunk, b):
            for j in range(CHUNK_BAGS):
                pltpu.make_async_copy(
                    table_hbm.at[_bag_idx(chunk, j)],
                    rows_b[b].at[pl.ds(j * GHIST, GHIST)],
                    sems[b]).wait()

        def compute(chunk, b):
            rb = rows_b[b]

            def bag_body(j, carry):
                rbase = j * GHIST

                def r_body(ri, accs):
                    out = list(accs)
                    for u in range(10):
                        row = rbase + ri * 10 + u
                        for dk in range(4):
                            out[dk] = out[dk] + rb[row, pl.ds(dk * 16, 16)]
                    return tuple(out)

                z = jnp.zeros((16,), jnp.float32)
                accs = lax.fori_loop(0, HIST // 10, r_body, (z, z, z, z))
                gbag = chunk * CHUNK_BAGS + j
                for dk in range(4):
                    acc_v[gbag, pl.ds(dk * 16, 16)] = accs[dk]
                return carry

            lax.fori_loop(0, CHUNK_BAGS, bag_body, 0)

        fire(0, 0)
        fire(1, 1)

        def step(c, carry):
            for b in range(2):
                chunk = 2 * c + b
                drain(chunk, b)
                compute(chunk, b)

                @pl.when(chunk < CHUNKS_PER_TILE - 2)
                def _():
                    fire(chunk + 2, b)
            return carry

        lax.fori_loop(0, CHUNKS_PER_TILE // 2, step, 0)
        pltpu.sync_copy(acc_v, out_hbm.at[pl.ds(bag0, BAGS_PER_TILE)])

    return sc_pool(xp, table)


def _tc_body(p_ref, w_ref, b_ref, o_ref):
    p = p_ref[:] * (1.0 / HIST)
    logits = lax.dot_general(p, w_ref[:], (((1,), (1,)), ((), ())),
                             preferred_element_type=jnp.float32)
    logits = logits + b_ref[:]
    m = jnp.max(logits, axis=1, keepdims=True)
    e = jnp.exp(logits - m)
    o_ref[:] = e / jnp.sum(e, axis=1, keepdims=True)


_TC_BLOCK = 1024


def _tc_dense(pooled, W, b2):
    return pl.pallas_call(
        _tc_body,
        grid=(BATCH // _TC_BLOCK,),
        in_specs=[
            pl.BlockSpec((_TC_BLOCK, EMBED_DIM), lambda i: (i, 0)),
            pl.BlockSpec((DENSE_OUT, EMBED_DIM), lambda i: (0, 0)),
            pl.BlockSpec((1, DENSE_OUT), lambda i: (0, 0)),
        ],
        out_specs=pl.BlockSpec((_TC_BLOCK, DENSE_OUT), lambda i: (i, 0)),
        out_shape=jax.ShapeDtypeStruct((BATCH, DENSE_OUT), jnp.float32),
    )(pooled, W, b2)


@jax.jit
def kernel(x, table, W, b):
    xp = jnp.pad(x.astype(jnp.int32), ((0, 0), (0, XPAD - HIST)))
    pooled = _sc_pool(xp, table)
    return _tc_dense(pooled, W, b.reshape(1, DENSE_OUT))


# speedup vs baseline: 1.0026x; 1.0026x over previous
import functools

import jax
import jax.numpy as jnp
from jax import lax
from jax.experimental import pallas as pl
from jax.experimental.pallas import tpu as pltpu
from jax.experimental.pallas import tpu_sc as plsc

NUM_EMBEDDINGS = 1000000
EMBED_DIM = 64
DENSE_OUT = 64
BATCH = 16384
HIST = 50

NC = 2
NS = 16
NW = NC * NS

BAGS_PER_TILE = BATCH // NW
CHUNK_BAGS = 4
CHUNKS_PER_TILE = BAGS_PER_TILE // CHUNK_BAGS
GHIST = 56
IDX_PER_CHUNK = CHUNK_BAGS * GHIST
XPAD = 128
SUPER_BAGS = 64
SUPER_PER_TILE = BAGS_PER_TILE // SUPER_BAGS
CHUNKS_PER_SUPER = SUPER_BAGS // CHUNK_BAGS


def _sc_pool(xp, table):
    mesh = plsc.VectorSubcoreMesh(core_axis_name="c", subcore_axis_name="s")

    @functools.partial(
        pl.kernel,
        mesh=mesh,
        compiler_params=pltpu.CompilerParams(use_tc_tiling_on_sc=False),
        out_type=jax.ShapeDtypeStruct((BATCH, XPAD), jnp.float32),
        scratch_types=[
            pltpu.VMEM((2, SUPER_BAGS, GHIST), jnp.int32),
            pltpu.VMEM((2, IDX_PER_CHUNK, EMBED_DIM), jnp.float32),
            pltpu.VMEM((BAGS_PER_TILE, XPAD), jnp.float32),
            pltpu.SemaphoreType.DMA,
            pltpu.SemaphoreType.DMA,
        ],
    )
    def sc_pool(x_hbm, table_hbm, out_hbm, idx_v, rows_v, acc_v, sem0, sem1):
        wid = lax.axis_index("s") * NC + lax.axis_index("c")
        bag0 = wid * BAGS_PER_TILE
        sems = (sem0, sem1)
        rows_b = (rows_v.at[0], rows_v.at[1])

        def _bag_idx(chunk, j):
            s = chunk // CHUNKS_PER_SUPER
            r = (chunk % CHUNKS_PER_SUPER) * CHUNK_BAGS + j
            return idx_v.at[s % 2, r]

        def fire(chunk, b):
            @pl.when(chunk % CHUNKS_PER_SUPER == 0)
            def _():
                s = chunk // CHUNKS_PER_SUPER
                pltpu.sync_copy(
                    x_hbm.at[pl.ds(bag0 + s * SUPER_BAGS, SUPER_BAGS),
                             pl.ds(0, GHIST)],
                    idx_v.at[s % 2])

            for j in range(CHUNK_BAGS):
                pltpu.async_copy(
                    table_hbm.at[_bag_idx(chunk, j)],
                    rows_b[b].at[pl.ds(j * GHIST, GHIST)],
                    sems[b])

        def drain(chunk, b):
            for j in range(CHUNK_BAGS):
                pltpu.make_async_copy(
                    table_hbm.at[_bag_idx(chunk, j)],
                    rows_b[b].at[pl.ds(j * GHIST, GHIST)],
                    sems[b]).wait()

        def compute(chunk, b):
            rb = rows_b[b]

            def bag_body(j, carry):
                rbase = j * GHIST

                def r_body(ri, accs):
                    out = list(accs)
                    for u in range(10):
                        row = rbase + ri * 10 + u
                        for dk in range(4):
                            out[dk] = out[dk] + rb[row, pl.ds(dk * 16, 16)]
                    return tuple(out)

                z = jnp.zeros((16,), jnp.float32)
                accs = lax.fori_loop(0, HIST // 10, r_body, (z, z, z, z))
                gbag = chunk * CHUNK_BAGS + j
                for dk in range(4):
                    acc_v[gbag, pl.ds(dk * 16, 16)] = accs[dk]
                return carry

            lax.fori_loop(0, CHUNK_BAGS, bag_body, 0)

        fire(0, 0)
        fire(1, 1)

        def step(c, carry):
            for b in range(2):
                chunk = 2 * c + b
                drain(chunk, b)
                compute(chunk, b)

                @pl.when(chunk < CHUNKS_PER_TILE - 2)
                def _():
                    fire(chunk + 2, b)
            return carry

        lax.fori_loop(0, CHUNKS_PER_TILE // 2, step, 0)
        pltpu.sync_copy(acc_v, out_hbm.at[pl.ds(bag0, BAGS_PER_TILE)])

    return sc_pool(xp, table)


def _tc_body(p_ref, w_ref, b_ref, o_ref):
    p = p_ref[:, :EMBED_DIM] * (1.0 / HIST)
    logits = lax.dot_general(p, w_ref[:], (((1,), (1,)), ((), ())),
                             preferred_element_type=jnp.float32)
    logits = logits + b_ref[:]
    m = jnp.max(logits, axis=1, keepdims=True)
    e = jnp.exp(logits - m)
    o_ref[:] = e / jnp.sum(e, axis=1, keepdims=True)


_TC_BLOCK = 1024


def _tc_dense(pooled, W, b2):
    return pl.pallas_call(
        _tc_body,
        grid=(BATCH // _TC_BLOCK,),
        in_specs=[
            pl.BlockSpec((_TC_BLOCK, XPAD), lambda i: (i, 0)),
            pl.BlockSpec((DENSE_OUT, EMBED_DIM), lambda i: (0, 0)),
            pl.BlockSpec((1, DENSE_OUT), lambda i: (0, 0)),
        ],
        out_specs=pl.BlockSpec((_TC_BLOCK, DENSE_OUT), lambda i: (i, 0)),
        out_shape=jax.ShapeDtypeStruct((BATCH, DENSE_OUT), jnp.float32),
    )(pooled, W, b2)


@jax.jit
def kernel(x, table, W, b):
    xp = jnp.pad(x.astype(jnp.int32), ((0, 0), (0, XPAD - HIST)))
    pooled = _sc_pool(xp, table)
    return _tc_dense(pooled, W, b.reshape(1, DENSE_OUT))


# speedup vs baseline: 3.4343x; 3.4253x over previous
import functools

import jax
import jax.numpy as jnp
from jax import lax
from jax.experimental import pallas as pl
from jax.experimental.pallas import tpu as pltpu
from jax.experimental.pallas import tpu_sc as plsc

NUM_EMBEDDINGS = 1000000
EMBED_DIM = 64
DENSE_OUT = 64
BATCH = 16384
HIST = 50

NC = 2
NS = 16
NW = NC * NS

BAGS_PER_TILE = BATCH // NW
CHUNK_BAGS = 4
CHUNKS_PER_TILE = BAGS_PER_TILE // CHUNK_BAGS
GHIST = 56
IDX_PER_CHUNK = CHUNK_BAGS * GHIST
XPAD = 128
SUPER_BAGS = 64
SUPER_PER_TILE = BAGS_PER_TILE // SUPER_BAGS
CHUNKS_PER_SUPER = SUPER_BAGS // CHUNK_BAGS


def _sc_pool(xp, table):
    mesh = plsc.VectorSubcoreMesh(core_axis_name="c", subcore_axis_name="s")

    @functools.partial(
        pl.kernel,
        mesh=mesh,
        compiler_params=pltpu.CompilerParams(use_tc_tiling_on_sc=False),
        out_type=jax.ShapeDtypeStruct((BATCH, XPAD), jnp.float32),
        scratch_types=[
            pltpu.VMEM((2, SUPER_BAGS, GHIST), jnp.int32),
            pltpu.VMEM((2, IDX_PER_CHUNK, EMBED_DIM), jnp.float32),
            pltpu.VMEM((BAGS_PER_TILE, XPAD), jnp.float32),
            pltpu.SemaphoreType.DMA,
            pltpu.SemaphoreType.DMA,
        ],
    )
    def sc_pool(x_hbm, table_hbm, out_hbm, idx_v, rows_v, acc_v, sem0, sem1):
        wid = lax.axis_index("s") * NC + lax.axis_index("c")
        bag0 = wid * BAGS_PER_TILE
        sems = (sem0, sem1)
        rows_b = (rows_v.at[0], rows_v.at[1])

        def _bag_idx(chunk, j):
            s = chunk // CHUNKS_PER_SUPER
            r = (chunk % CHUNKS_PER_SUPER) * CHUNK_BAGS + j
            return idx_v.at[s % 2, r]

        def fire(chunk, b):
            @pl.when(chunk % CHUNKS_PER_SUPER == 0)
            def _():
                s = chunk // CHUNKS_PER_SUPER
                pltpu.sync_copy(
                    x_hbm.at[pl.ds(bag0 + s * SUPER_BAGS, SUPER_BAGS),
                             pl.ds(0, GHIST)],
                    idx_v.at[s % 2])

            for j in range(CHUNK_BAGS):
                pltpu.async_copy(
                    table_hbm.at[_bag_idx(chunk, j)],
                    rows_b[b].at[pl.ds(j * GHIST, GHIST)],
                    sems[b])

        def drain(chunk, b):
            for j in range(CHUNK_BAGS):
                pltpu.make_async_copy(
                    table_hbm.at[_bag_idx(chunk, j)],
                    rows_b[b].at[pl.ds(j * GHIST, GHIST)],
                    sems[b]).wait()

        def compute(chunk, b):
            rb = rows_b[b]

            def bag_body(j, carry):
                rbase = j * GHIST

                def r_body(ri, accs):
                    out = list(accs)
                    for u in range(10):
                        row = rbase + ri * 10 + u
                        for dk in range(4):
                            out[dk] = out[dk] + rb[row, pl.ds(dk * 16, 16)]
                    return tuple(out)

                z = jnp.zeros((16,), jnp.float32)
                accs = lax.fori_loop(0, HIST // 10, r_body, (z, z, z, z))
                gbag = chunk * CHUNK_BAGS + j
                for dk in range(4):
                    acc_v[gbag, pl.ds(dk * 16, 16)] = accs[dk]
                return carry

            lax.fori_loop(0, CHUNK_BAGS, bag_body, 0)

        fire(0, 0)
        fire(1, 1)

        def step(c, carry):
            for b in range(2):
                chunk = 2 * c + b
                drain(chunk, b)
                compute(chunk, b)

                @pl.when(chunk < CHUNKS_PER_TILE - 2)
                def _():
                    fire(chunk + 2, b)
            return carry

        lax.fori_loop(0, CHUNKS_PER_TILE // 2, step, 0)
        pltpu.sync_copy(acc_v, out_hbm.at[pl.ds(bag0, BAGS_PER_TILE)])

    return sc_pool(xp, table)


def _tc_body(p_ref, w_ref, b_ref, o_ref):
    p = p_ref[:, :EMBED_DIM] * (1.0 / HIST)
    logits = lax.dot_general(p, w_ref[:], (((1,), (1,)), ((), ())),
                             preferred_element_type=jnp.float32)
    logits = logits + b_ref[:]
    m = jnp.max(logits, axis=1, keepdims=True)
    e = jnp.exp(logits - m)
    o_ref[:] = e / jnp.sum(e, axis=1, keepdims=True)


_TC_BLOCK = 1024


def _tc_dense(pooled, W, b2):
    return pl.pallas_call(
        _tc_body,
        grid=(BATCH // _TC_BLOCK,),
        in_specs=[
            pl.BlockSpec((_TC_BLOCK, XPAD), lambda i: (i, 0)),
            pl.BlockSpec((DENSE_OUT, EMBED_DIM), lambda i: (0, 0)),
            pl.BlockSpec((1, DENSE_OUT), lambda i: (0, 0)),
        ],
        out_specs=pl.BlockSpec((_TC_BLOCK, DENSE_OUT), lambda i: (i, 0)),
        out_shape=jax.ShapeDtypeStruct((BATCH, DENSE_OUT), jnp.float32),
    )(pooled, W, b2)


@jax.jit
def kernel(x, table, W, b):
    xi = x.astype(jnp.int32)
    xp = jnp.pad(jnp.concatenate([xi, xi[:, :GHIST - HIST]], axis=1),
                 ((0, 0), (0, XPAD - GHIST)))
    pooled = _sc_pool(xp, table)
    return _tc_dense(pooled, W, b.reshape(1, DENSE_OUT))


# speedup vs baseline: 4.7932x; 1.3957x over previous
import functools

import jax
import jax.numpy as jnp
from jax import lax
from jax.experimental import pallas as pl
from jax.experimental.pallas import tpu as pltpu
from jax.experimental.pallas import tpu_sc as plsc

NUM_EMBEDDINGS = 1000000
EMBED_DIM = 64
DENSE_OUT = 64
BATCH = 16384
HIST = 50

NC = 2
NS = 16
NW = NC * NS

BAGS_PER_TILE = BATCH // NW
CHUNK_BAGS = 4
CHUNKS_PER_TILE = BAGS_PER_TILE // CHUNK_BAGS
GHIST = 56
IDX_PER_CHUNK = CHUNK_BAGS * GHIST
XPAD = 128
SUPER_BAGS = 64
SUPER_PER_TILE = BAGS_PER_TILE // SUPER_BAGS
CHUNKS_PER_SUPER = SUPER_BAGS // CHUNK_BAGS


_TR_BLOCK = 8192
_TR_GRID = -(-NUM_EMBEDDINGS // _TR_BLOCK)


def _tr_body(t_ref, o_ref):
    eye = jnp.eye(EMBED_DIM, dtype=jnp.float32)
    o_ref[:, :EMBED_DIM] = lax.dot_general(
        t_ref[:], eye, (((0,), (0,)), ((), ())),
        preferred_element_type=jnp.float32)


def _tc_repack(tableT):
    return pl.pallas_call(
        _tr_body,
        grid=(_TR_GRID,),
        in_specs=[pl.BlockSpec((EMBED_DIM, _TR_BLOCK), lambda i: (0, i))],
        out_specs=pl.BlockSpec((_TR_BLOCK, XPAD), lambda i: (i, 0)),
        out_shape=jax.ShapeDtypeStruct((NUM_EMBEDDINGS, XPAD), jnp.float32),
    )(tableT)


def _sc_pool(xp, table):
    mesh = plsc.VectorSubcoreMesh(core_axis_name="c", subcore_axis_name="s")

    @functools.partial(
        pl.kernel,
        mesh=mesh,
        compiler_params=pltpu.CompilerParams(use_tc_tiling_on_sc=False),
        out_type=jax.ShapeDtypeStruct((BATCH, EMBED_DIM), jnp.float32),
        scratch_types=[
            pltpu.VMEM((2, SUPER_BAGS, GHIST), jnp.int32),
            pltpu.VMEM((2, IDX_PER_CHUNK, XPAD), jnp.float32),
            pltpu.VMEM((BAGS_PER_TILE, EMBED_DIM), jnp.float32),
            pltpu.SemaphoreType.DMA,
            pltpu.SemaphoreType.DMA,
        ],
    )
    def sc_pool(x_hbm, table_hbm, out_hbm, idx_v, rows_v, acc_v, sem0, sem1):
        wid = lax.axis_index("s") * NC + lax.axis_index("c")
        bag0 = wid * BAGS_PER_TILE
        sems = (sem0, sem1)
        rows_b = (rows_v.at[0], rows_v.at[1])

        def _bag_idx(chunk, j):
            s = chunk // CHUNKS_PER_SUPER
            r = (chunk % CHUNKS_PER_SUPER) * CHUNK_BAGS + j
            return idx_v.at[s % 2, r]

        def _src(chunk, j):
            return table_hbm.at[_bag_idx(chunk, j)]

        def fire(chunk, b):
            @pl.when(chunk % CHUNKS_PER_SUPER == 0)
            def _():
                s = chunk // CHUNKS_PER_SUPER
                pltpu.sync_copy(
                    x_hbm.at[pl.ds(bag0 + s * SUPER_BAGS, SUPER_BAGS),
                             pl.ds(0, GHIST)],
                    idx_v.at[s % 2])

            for j in range(CHUNK_BAGS):
                pltpu.async_copy(
                    _src(chunk, j),
                    rows_b[b].at[pl.ds(j * GHIST, GHIST)],
                    sems[b])

        def drain(chunk, b):
            for j in range(CHUNK_BAGS):
                pltpu.make_async_copy(
                    _src(chunk, j),
                    rows_b[b].at[pl.ds(j * GHIST, GHIST)],
                    sems[b]).wait()

        def compute(chunk, b):
            rb = rows_b[b]

            def bag_body(j, carry):
                rbase = j * GHIST

                def r_body(ri, accs):
                    out = list(accs)
                    for u in range(10):
                        row = rbase + ri * 10 + u
                        for dk in range(4):
                            out[dk] = out[dk] + rb[row, pl.ds(dk * 16, 16)]
                    return tuple(out)

                z = jnp.zeros((16,), jnp.float32)
                accs = lax.fori_loop(0, HIST // 10, r_body, (z, z, z, z))
                gbag = chunk * CHUNK_BAGS + j
                for dk in range(4):
                    acc_v[gbag, pl.ds(dk * 16, 16)] = accs[dk]
                return carry

            lax.fori_loop(0, CHUNK_BAGS, bag_body, 0)

        fire(0, 0)
        fire(1, 1)

        def step(c, carry):
            for b in range(2):
                chunk = 2 * c + b
                drain(chunk, b)
                compute(chunk, b)

                @pl.when(chunk < CHUNKS_PER_TILE - 2)
                def _():
                    fire(chunk + 2, b)
            return carry

        lax.fori_loop(0, CHUNKS_PER_TILE // 2, step, 0)
        pltpu.sync_copy(acc_v, out_hbm.at[pl.ds(bag0, BAGS_PER_TILE)])

    return sc_pool(xp, table)


def _tc_body(p_ref, w_ref, b_ref, o_ref):
    p = p_ref[:] * (1.0 / HIST)
    logits = lax.dot_general(p, w_ref[:], (((1,), (1,)), ((), ())),
                             preferred_element_type=jnp.float32)
    logits = logits + b_ref[:]
    m = jnp.max(logits, axis=1, keepdims=True)
    e = jnp.exp(logits - m)
    o_ref[:] = e / jnp.sum(e, axis=1, keepdims=True)


_TC_BLOCK = 1024


def _tc_dense(pooled, W, b2):
    return pl.pallas_call(
        _tc_body,
        grid=(BATCH // _TC_BLOCK,),
        in_specs=[
            pl.BlockSpec((_TC_BLOCK, EMBED_DIM), lambda i: (i, 0)),
            pl.BlockSpec((DENSE_OUT, EMBED_DIM), lambda i: (0, 0)),
            pl.BlockSpec((1, DENSE_OUT), lambda i: (0, 0)),
        ],
        out_specs=pl.BlockSpec((_TC_BLOCK, DENSE_OUT), lambda i: (i, 0)),
        out_shape=jax.ShapeDtypeStruct((BATCH, DENSE_OUT), jnp.float32),
    )(pooled, W, b2)


@jax.jit
def kernel(x, table, W, b):
    xi = x.astype(jnp.int32)
    xp = jnp.pad(jnp.concatenate([xi, xi[:, :GHIST - HIST]], axis=1),
                 ((0, 0), (0, XPAD - GHIST)))
    tp = _tc_repack(table.T)
    pooled = _sc_pool(xp, tp)
    return _tc_dense(pooled, W, b.reshape(1, DENSE_OUT))
